# baseline (speedup 1.0000x reference)
import functools

import jax
import jax.numpy as jnp
from jax import lax
from jax.experimental import pallas as pl
from jax.experimental.pallas import tpu as pltpu
from jax.experimental.pallas import tpu_sc as plsc

VOCAB = 100000
EMBED = 64
BATCH = 1024

_NC, _NS = 2, 16
_NW = _NC * _NS
_B_PER_W = BATCH // _NW

_mesh = plsc.VectorSubcoreMesh(core_axis_name="c", subcore_axis_name="s")


@functools.partial(
    pl.kernel,
    mesh=_mesh,
    out_type=jax.ShapeDtypeStruct((BATCH, EMBED), jnp.float32),
    scratch_types=[
        pltpu.VMEM((_B_PER_W,), jnp.int32),
        pltpu.VMEM((_B_PER_W, EMBED), jnp.float32),
        pltpu.SemaphoreType.DMA,
    ],
    compiler_params=pltpu.CompilerParams(use_tc_tiling_on_sc=False),
)
def _sc_gather(table_hbm, idx_hbm, out_hbm, idx_v, rows_v, sem):
    wid = lax.axis_index("s") * _NC + lax.axis_index("c")
    base = wid * _B_PER_W
    pltpu.sync_copy(idx_hbm.at[pl.ds(base, _B_PER_W)], idx_v)
    pltpu.async_copy(table_hbm.at[idx_v], rows_v, sem).wait()
    pltpu.sync_copy(rows_v, out_hbm.at[pl.ds(base, _B_PER_W)])


_VB = 2048
_NFULL = VOCAB // _VB
_TAIL = VOCAB - _NFULL * _VB
_NSTEP = _NFULL + 1
_NBUF = 3
_NSPLIT = 4
_RS = BATCH // _NSPLIT


def _mm_body(e_ref, w_ref, b_ref, out_ref, obuf, tailbuf, sem):
    j = pl.program_id(0)
    slot = lax.rem(j, _NBUF)

    def _full_copy(s, sl, col):
        return pltpu.make_async_copy(
            obuf.at[sl, pl.ds(s * _RS, _RS)],
            out_ref.at[pl.ds(s * _RS, _RS), pl.ds(col, _VB)],
            sem.at[sl, s],
        )

    def _tail_copy(s, sl):
        return pltpu.make_async_copy(
            tailbuf.at[pl.ds(s * _RS, _RS)],
            out_ref.at[pl.ds(s * _RS, _RS), pl.ds(_NFULL * _VB, _TAIL)],
            sem.at[sl, s],
        )

    @pl.when(j >= _NBUF)
    def _():
        for s in range(_NSPLIT):
            _full_copy(s, slot, 0).wait()

    acc = lax.dot_general(
        e_ref[...], w_ref[...],
        dimension_numbers=(((1,), (1,)), ((), ())),
        preferred_element_type=jnp.float32,
    )
    biased = acc + b_ref[...]

    @pl.when(j < _NFULL)
    def _():
        obuf[slot] = biased
        for s in range(_NSPLIT):
            _full_copy(s, slot, j * _VB).start()

    @pl.when(j == _NSTEP - 1)
    def _():
        tailbuf[...] = biased[:, :_TAIL]
        for s in range(_NSPLIT):
            _tail_copy(s, slot).start()
        for dj in range(_NSTEP - _NBUF, _NSTEP - 1):
            for s in range(_NSPLIT):
                _full_copy(s, dj % _NBUF, 0).wait()
        for s in range(_NSPLIT):
            _tail_copy(s, slot).wait()


def _projection(embeds, W, b2d):
    return pl.pallas_call(
        _mm_body,
        grid=(_NSTEP,),
        in_specs=[
            pl.BlockSpec((BATCH, EMBED), lambda j: (0, 0)),
            pl.BlockSpec((_VB, EMBED), lambda j: (j, 0)),
            pl.BlockSpec((1, _VB), lambda j: (0, j)),
        ],
        out_specs=pl.BlockSpec(memory_space=pl.ANY),
        out_shape=jax.ShapeDtypeStruct((BATCH, VOCAB), jnp.float32),
        scratch_shapes=[
            pltpu.VMEM((_NBUF, BATCH, _VB), jnp.float32),
            pltpu.VMEM((BATCH, _TAIL), jnp.float32),
            pltpu.SemaphoreType.DMA((_NBUF, _NSPLIT)),
        ],
    )(embeds, W, b2d)


def kernel(center_words, embedding, W, b):
    idx = center_words.astype(jnp.int32)
    embeds = _sc_gather(embedding, idx)
    return _projection(embeds, W, b.reshape(1, VOCAB))


# speedup vs baseline: 1.0010x; 1.0010x over previous
import functools

import jax
import jax.numpy as jnp
from jax import lax
from jax.experimental import pallas as pl
from jax.experimental.pallas import tpu as pltpu
from jax.experimental.pallas import tpu_sc as plsc

VOCAB = 100000
EMBED = 64
BATCH = 1024

_NC, _NS = 2, 16
_NW = _NC * _NS
_B_PER_W = BATCH // _NW

_mesh = plsc.VectorSubcoreMesh(core_axis_name="c", subcore_axis_name="s")


@functools.partial(
    pl.kernel,
    mesh=_mesh,
    out_type=jax.ShapeDtypeStruct((BATCH, EMBED), jnp.float32),
    scratch_types=[
        pltpu.VMEM((_B_PER_W,), jnp.int32),
        pltpu.VMEM((_B_PER_W, EMBED), jnp.float32),
        pltpu.SemaphoreType.DMA,
    ],
    compiler_params=pltpu.CompilerParams(use_tc_tiling_on_sc=False),
)
def _sc_gather(table_hbm, idx_hbm, out_hbm, idx_v, rows_v, sem):
    wid = lax.axis_index("s") * _NC + lax.axis_index("c")
    base = wid * _B_PER_W
    pltpu.sync_copy(idx_hbm.at[pl.ds(base, _B_PER_W)], idx_v)
    pltpu.async_copy(table_hbm.at[idx_v], rows_v, sem).wait()
    pltpu.sync_copy(rows_v, out_hbm.at[pl.ds(base, _B_PER_W)])


_VB = 2048
_NFULL = VOCAB // _VB
_TAIL = VOCAB - _NFULL * _VB
_MID = (_TAIL // 128) * 128
_LAST = _TAIL - _MID
_NSTEP = _NFULL + 1
_NBUF = 3
_NSPLIT = 4
_RS = BATCH // _NSPLIT


def _mm_body(e_ref, w_ref, b_ref, out_ref, obuf, midbuf, lastbuf, sem, sem_last):
    j = pl.program_id(0)
    slot = lax.rem(j, _NBUF)

    def _full_copy(s, sl, col):
        return pltpu.make_async_copy(
            obuf.at[sl, pl.ds(s * _RS, _RS)],
            out_ref.at[pl.ds(s * _RS, _RS), pl.ds(col, _VB)],
            sem.at[sl, s],
        )

    def _mid_copy(s, sl):
        return pltpu.make_async_copy(
            midbuf.at[pl.ds(s * _RS, _RS)],
            out_ref.at[pl.ds(s * _RS, _RS), pl.ds(_NFULL * _VB, _MID)],
            sem.at[sl, s],
        )

    def _last_copy():
        return pltpu.make_async_copy(
            lastbuf,
            out_ref.at[:, pl.ds(_NFULL * _VB + _MID, _LAST)],
            sem_last,
        )

    @pl.when(j >= _NBUF)
    def _():
        for s in range(_NSPLIT):
            _full_copy(s, slot, 0).wait()

    acc = lax.dot_general(
        e_ref[...], w_ref[...],
        dimension_numbers=(((1,), (1,)), ((), ())),
        preferred_element_type=jnp.float32,
    )
    biased = acc + b_ref[...]

    @pl.when(j < _NFULL)
    def _():
        obuf[slot] = biased
        for s in range(_NSPLIT):
            _full_copy(s, slot, j * _VB).start()

    @pl.when(j == _NSTEP - 1)
    def _():
        midbuf[...] = biased[:, :_MID]
        lastbuf[...] = biased[:, _MID:_TAIL]
        for s in range(_NSPLIT):
            _mid_copy(s, slot).start()
        _last_copy().start()
        for dj in range(_NSTEP - _NBUF, _NSTEP - 1):
            for s in range(_NSPLIT):
                _full_copy(s, dj % _NBUF, 0).wait()
        for s in range(_NSPLIT):
            _mid_copy(s, slot).wait()
        _last_copy().wait()


def _projection(embeds, W, b2d):
    return pl.pallas_call(
        _mm_body,
        grid=(_NSTEP,),
        in_specs=[
            pl.BlockSpec((BATCH, EMBED), lambda j: (0, 0)),
            pl.BlockSpec((_VB, EMBED), lambda j: (j, 0)),
            pl.BlockSpec((1, _VB), lambda j: (0, j)),
        ],
        out_specs=pl.BlockSpec(memory_space=pl.ANY),
        out_shape=jax.ShapeDtypeStruct((BATCH, VOCAB), jnp.float32),
        scratch_shapes=[
            pltpu.VMEM((_NBUF, BATCH, _VB), jnp.float32),
            pltpu.VMEM((BATCH, _MID), jnp.float32),
            pltpu.VMEM((BATCH, _LAST), jnp.float32),
            pltpu.SemaphoreType.DMA((_NBUF, _NSPLIT)),
            pltpu.SemaphoreType.DMA,
        ],
    )(embeds, W, b2d)


def kernel(center_words, embedding, W, b):
    idx = center_words.astype(jnp.int32)
    embeds = _sc_gather(embedding, idx)
    return _projection(embeds, W, b.reshape(1, VOCAB))


# speedup vs baseline: 1.0913x; 1.0902x over previous
import functools

import jax
import jax.numpy as jnp
from jax import lax
from jax.experimental import pallas as pl
from jax.experimental.pallas import tpu as pltpu
from jax.experimental.pallas import tpu_sc as plsc

VOCAB = 100000
EMBED = 64
BATCH = 1024

_NC, _NS = 2, 16
_NW = _NC * _NS
_B_PER_W = BATCH // _NW

_mesh = plsc.VectorSubcoreMesh(core_axis_name="c", subcore_axis_name="s")


@functools.partial(
    pl.kernel,
    mesh=_mesh,
    out_type=jax.ShapeDtypeStruct((BATCH, EMBED), jnp.float32),
    scratch_types=[
        pltpu.VMEM((_B_PER_W,), jnp.int32),
        pltpu.VMEM((_B_PER_W, EMBED), jnp.float32),
        pltpu.SemaphoreType.DMA,
    ],
    compiler_params=pltpu.CompilerParams(use_tc_tiling_on_sc=False),
)
def _sc_gather(table_hbm, idx_hbm, out_hbm, idx_v, rows_v, sem):
    wid = lax.axis_index("s") * _NC + lax.axis_index("c")
    base = wid * _B_PER_W
    pltpu.sync_copy(idx_hbm.at[pl.ds(base, _B_PER_W)], idx_v)
    pltpu.async_copy(table_hbm.at[idx_v], rows_v, sem).wait()
    pltpu.sync_copy(rows_v, out_hbm.at[pl.ds(base, _B_PER_W)])


_VB = 2048
_NFULL = VOCAB // _VB
_TAIL = VOCAB - _NFULL * _VB
_MID = (_TAIL // 128) * 128
_LAST = _TAIL - _MID
_NSTEP = _NFULL + 1
_NBUF = 3
_NSPLIT = 4
_RS = BATCH // _NSPLIT


def _mm_body(e_hbm, wt_hbm, b_hbm, out_ref,
             ebuf, wtbuf, wtail, bbuf, obuf, lastbuf, lsem, sem, sem_last):
    j = pl.program_id(0)
    slot = lax.rem(j, _NBUF)

    @pl.when(j == 0)
    def _():
        copies = [
            pltpu.make_async_copy(e_hbm, ebuf, lsem),
            pltpu.make_async_copy(b_hbm, bbuf, lsem),
            pltpu.make_async_copy(wt_hbm.at[:, pl.ds(_NFULL * _VB, _TAIL)],
                                  wtail, lsem),
        ] + [
            pltpu.make_async_copy(wt_hbm.at[:, pl.ds(k * _VB, _VB)],
                                  wtbuf.at[k], lsem)
            for k in range(_NFULL)
        ]
        for c in copies:
            c.start()
        for c in copies:
            c.wait()

    def _full_copy(s, sl, col):
        return pltpu.make_async_copy(
            obuf.at[sl, pl.ds(s * _RS, _RS)],
            out_ref.at[pl.ds(s * _RS, _RS), pl.ds(col, _VB)],
            sem.at[sl, s],
        )

    def _mid_copy(s, sl):
        return pltpu.make_async_copy(
            obuf.at[sl, pl.ds(s * _RS, _RS), pl.ds(0, _MID)],
            out_ref.at[pl.ds(s * _RS, _RS), pl.ds(_NFULL * _VB, _MID)],
            sem.at[sl, s],
        )

    def _last_copy():
        return pltpu.make_async_copy(
            lastbuf,
            out_ref.at[:, pl.ds(_NFULL * _VB + _MID, _LAST)],
            sem_last,
        )

    @pl.when(j >= _NBUF)
    def _():
        for s in range(_NSPLIT):
            _full_copy(s, slot, 0).wait()

    e = ebuf[...]

    @pl.when(j < _NFULL)
    def _():
        acc = lax.dot_general(
            e, wtbuf[j],
            dimension_numbers=(((1,), (0,)), ((), ())),
            preferred_element_type=jnp.float32,
        )
        obuf[slot] = acc + bbuf[j]
        for s in range(_NSPLIT):
            _full_copy(s, slot, j * _VB).start()

    @pl.when(j == _NSTEP - 1)
    def _():
        acc_mid = lax.dot_general(
            e, wtail[:, :_MID],
            dimension_numbers=(((1,), (0,)), ((), ())),
            preferred_element_type=jnp.float32,
        )
        obuf[slot, :, : _MID] = acc_mid + bbuf[_NFULL, :, :_MID]
        acc_last = lax.dot_general(
            e, wtail[:, _MID:],
            dimension_numbers=(((1,), (0,)), ((), ())),
            preferred_element_type=jnp.float32,
        )
        lastbuf[...] = acc_last + bbuf[_NFULL, :, _MID:_TAIL]
        for s in range(_NSPLIT):
            _mid_copy(s, slot).start()
        _last_copy().start()
        for dj in range(_NSTEP - _NBUF, _NSTEP - 1):
            for s in range(_NSPLIT):
                _full_copy(s, dj % _NBUF, 0).wait()
        for s in range(_NSPLIT):
            _mid_copy(s, slot).wait()
        _last_copy().wait()


def _projection(embeds, WT, bpad):
    return pl.pallas_call(
        _mm_body,
        grid=(_NSTEP,),
        in_specs=[
            pl.BlockSpec(memory_space=pl.ANY),
            pl.BlockSpec(memory_space=pl.ANY),
            pl.BlockSpec(memory_space=pl.ANY),
        ],
        out_specs=pl.BlockSpec(memory_space=pl.ANY),
        out_shape=jax.ShapeDtypeStruct((BATCH, VOCAB), jnp.float32),
        scratch_shapes=[
            pltpu.VMEM((BATCH, EMBED), jnp.float32),
            pltpu.VMEM((_NFULL, EMBED, _VB), jnp.float32),
            pltpu.VMEM((EMBED, _TAIL), jnp.float32),
            pltpu.VMEM((_NSTEP, 1, _VB), jnp.float32),
            pltpu.VMEM((_NBUF, BATCH, _VB), jnp.float32),
            pltpu.VMEM((BATCH, _LAST), jnp.float32),
            pltpu.SemaphoreType.DMA,
            pltpu.SemaphoreType.DMA((_NBUF, _NSPLIT)),
            pltpu.SemaphoreType.DMA,
        ],
    )(embeds, WT, bpad)


def kernel(center_words, embedding, W, b):
    idx = center_words.astype(jnp.int32)
    embeds = _sc_gather(embedding, idx)
    bpad = jnp.pad(b, (0, _NSTEP * _VB - VOCAB)).reshape(_NSTEP, 1, _VB)
    return _projection(embeds, W.T, bpad)
